# Initial kernel scaffold; baseline (speedup 1.0000x reference)
import functools

import jax
import jax.numpy as jnp
from jax.experimental import pallas as pl

_PERCENT_T = 0.02
_BISECT_ITERS = 24
_ROWS_PER_BLOCK = 8


def _stats_kernel(x_ref, out_ref, *, k):
    x = x_ref[...]
    rowmax = jnp.max(x, axis=1, keepdims=True)
    rowmin = jnp.min(x, axis=1, keepdims=True)
    lo = rowmin - 1.0
    hi = rowmax
    kf = jnp.float32(k)

    def body(_, carry):
        lo, hi = carry
        mid = 0.5 * (lo + hi)
        m = jnp.sum((x > mid).astype(jnp.float32), axis=1, keepdims=True)
        pred = m >= kf
        lo = jnp.where(pred, mid, lo)
        hi = jnp.where(pred, hi, mid)
        return lo, hi

    lo, hi = jax.lax.fori_loop(0, _BISECT_ITERS, body, (lo, hi))
    mask = x > hi
    m_hi = jnp.sum(mask.astype(jnp.float32), axis=1, keepdims=True)
    s_hi = jnp.sum(jnp.where(mask, x, 0.0), axis=1, keepdims=True)
    sum_est = s_hi + (kf - m_hi) * 0.5 * (lo + hi)
    avg = sum_est / kf
    out_ref[...] = jnp.concatenate([avg, rowmax], axis=1)


def _mlp_kernel(avg_ref, max_ref, w1_ref, b1_ref, w2_ref, b2_ref, scale_ref):
    w1 = w1_ref[...]
    b1 = b1_ref[...]
    w2 = w2_ref[...]
    b2 = b2_ref[...]

    def mlp(p):
        h = jnp.dot(p, w1.T, preferred_element_type=jnp.float32) + b1
        h = jnp.maximum(h, 0.0)
        return jnp.dot(h, w2.T, preferred_element_type=jnp.float32) + b2

    att = mlp(avg_ref[...]) + mlp(max_ref[...])
    scale_ref[...] = jax.nn.sigmoid(att)


def _scale_kernel(x_ref, s_ref, o_ref):
    o_ref[...] = x_ref[...] * s_ref[...]


def kernel(x, W1, b1, W2, b2):
    B, C, H, Wd = x.shape
    N = H * Wd
    R = B * C
    k = int(round(N * _PERCENT_T))
    x2 = x.reshape(R, N)

    rows = _ROWS_PER_BLOCK
    pools = pl.pallas_call(
        functools.partial(_stats_kernel, k=k),
        grid=(R // rows,),
        in_specs=[pl.BlockSpec((rows, N), lambda i: (i, 0))],
        out_specs=pl.BlockSpec((rows, 2), lambda i: (i, 0)),
        out_shape=jax.ShapeDtypeStruct((R, 2), jnp.float32),
    )(x2)

    avg_pool = pools[:, 0].reshape(B, C)
    max_pool = pools[:, 1].reshape(B, C)

    scale = pl.pallas_call(
        _mlp_kernel,
        out_shape=jax.ShapeDtypeStruct((B, C), jnp.float32),
    )(avg_pool, max_pool, W1, b1.reshape(1, -1), W2, b2.reshape(1, -1))

    scale2 = scale.reshape(R, 1)
    blk_n = 6144
    y = pl.pallas_call(
        _scale_kernel,
        grid=(R // rows, N // blk_n),
        in_specs=[
            pl.BlockSpec((rows, blk_n), lambda i, j: (i, j)),
            pl.BlockSpec((rows, 1), lambda i, j: (i, 0)),
        ],
        out_specs=pl.BlockSpec((rows, blk_n), lambda i, j: (i, j)),
        out_shape=jax.ShapeDtypeStruct((R, N), jnp.float32),
    )(x2, scale2)

    return y.reshape(B, C, H, Wd)


# speedup vs baseline: 20.4509x; 20.4509x over previous
import functools

import jax
import jax.numpy as jnp
from jax.experimental import pallas as pl

_PERCENT_T = 0.02
_BISECT_ITERS = 24
_ROWS_PER_BLOCK = 8


def _stats_kernel(x_ref, out_ref, *, k):
    x = x_ref[...]
    rowmax = jnp.max(x, axis=1, keepdims=True)
    rowmin = jnp.min(x, axis=1, keepdims=True)
    lo = rowmin - 1.0
    hi = rowmax
    kf = jnp.float32(k)

    def body(_, carry):
        lo, hi = carry
        mid = 0.5 * (lo + hi)
        m = jnp.sum((x > mid).astype(jnp.float32), axis=1, keepdims=True)
        pred = m >= kf
        lo = jnp.where(pred, mid, lo)
        hi = jnp.where(pred, hi, mid)
        return lo, hi

    lo, hi = jax.lax.fori_loop(0, _BISECT_ITERS, body, (lo, hi))
    mask = x > hi
    m_hi = jnp.sum(mask.astype(jnp.float32), axis=1, keepdims=True)
    s_hi = jnp.sum(jnp.where(mask, x, 0.0), axis=1, keepdims=True)
    sum_est = s_hi + (kf - m_hi) * 0.5 * (lo + hi)
    avg = sum_est / kf
    out_ref[...] = jnp.concatenate([avg, rowmax], axis=1)


def _mlp_kernel(avg_ref, max_ref, w1_ref, b1_ref, w2_ref, b2_ref, scale_ref):
    w1 = w1_ref[...]
    b1 = b1_ref[...]
    w2 = w2_ref[...]
    b2 = b2_ref[...]

    def mlp(p):
        h = jnp.dot(p, w1.T, preferred_element_type=jnp.float32) + b1
        h = jnp.maximum(h, 0.0)
        return jnp.dot(h, w2.T, preferred_element_type=jnp.float32) + b2

    att = mlp(avg_ref[...]) + mlp(max_ref[...])
    scale_ref[...] = jax.nn.sigmoid(att)


def _scale_kernel(x_ref, s_ref, o_ref):
    o_ref[...] = x_ref[...] * s_ref[...]


def kernel(x, W1, b1, W2, b2):
    B, C, H, Wd = x.shape
    N = H * Wd
    R = B * C
    k = int(round(N * _PERCENT_T))
    x2 = x.reshape(R, N)

    rows = _ROWS_PER_BLOCK
    pools = pl.pallas_call(
        functools.partial(_stats_kernel, k=k),
        grid=(R // rows,),
        in_specs=[pl.BlockSpec((rows, N), lambda i: (i, 0))],
        out_specs=pl.BlockSpec((rows, 2), lambda i: (i, 0)),
        out_shape=jax.ShapeDtypeStruct((R, 2), jnp.float32),
    )(x2)

    avg_pool = pools[:, 0].reshape(B, C)
    max_pool = pools[:, 1].reshape(B, C)

    scale = pl.pallas_call(
        _mlp_kernel,
        out_shape=jax.ShapeDtypeStruct((B, C), jnp.float32),
    )(avg_pool, max_pool, W1, b1.reshape(1, -1), W2, b2.reshape(1, -1))

    scale2 = scale.reshape(R, 1)
    blk_n = 6144 if N % 6144 == 0 else N
    y = pl.pallas_call(
        _scale_kernel,
        grid=(R // rows, N // blk_n),
        in_specs=[
            pl.BlockSpec((rows, blk_n), lambda i, j: (i, j)),
            pl.BlockSpec((rows, 1), lambda i, j: (i, 0)),
        ],
        out_specs=pl.BlockSpec((rows, blk_n), lambda i, j: (i, j)),
        out_shape=jax.ShapeDtypeStruct((R, N), jnp.float32),
    )(x2, scale2)

    return y.reshape(B, C, H, Wd)


# speedup vs baseline: 53.8304x; 2.6322x over previous
import functools

import jax
import jax.numpy as jnp
from jax.experimental import pallas as pl

_PERCENT_T = 0.02
_BISECT_ITERS = 24
_ROWS_PER_BLOCK = 8
_H_BLK = 96


def _stats_kernel(x_ref, out_ref, *, k):
    x = x_ref[...]
    rowmax = jnp.max(x, axis=(1, 2), keepdims=True)
    rowmin = jnp.min(x, axis=(1, 2), keepdims=True)
    lo = rowmin - 1.0
    hi = rowmax
    kf = jnp.float32(k)

    def body(_, carry):
        lo, hi = carry
        mid = 0.5 * (lo + hi)
        m = jnp.sum((x > mid).astype(jnp.float32), axis=(1, 2), keepdims=True)
        pred = m >= kf
        lo = jnp.where(pred, mid, lo)
        hi = jnp.where(pred, hi, mid)
        return lo, hi

    lo, hi = jax.lax.fori_loop(0, _BISECT_ITERS, body, (lo, hi))
    mask = x > hi
    m_hi = jnp.sum(mask.astype(jnp.float32), axis=(1, 2), keepdims=True)
    s_hi = jnp.sum(jnp.where(mask, x, 0.0), axis=(1, 2), keepdims=True)
    sum_est = s_hi + (kf - m_hi) * 0.5 * (lo + hi)
    avg = sum_est / kf
    out_ref[...] = jnp.concatenate([avg, rowmax], axis=2)[:, 0, :]


def _mlp_kernel(avg_ref, max_ref, w1_ref, b1_ref, w2_ref, b2_ref, scale_ref):
    w1 = w1_ref[...]
    b1 = b1_ref[...]
    w2 = w2_ref[...]
    b2 = b2_ref[...]

    def mlp(p):
        h = jnp.dot(p, w1.T, preferred_element_type=jnp.float32) + b1
        h = jnp.maximum(h, 0.0)
        return jnp.dot(h, w2.T, preferred_element_type=jnp.float32) + b2

    att = mlp(avg_ref[...]) + mlp(max_ref[...])
    scale_ref[...] = jax.nn.sigmoid(att)


def _scale_kernel(x_ref, s_ref, o_ref):
    o_ref[...] = x_ref[...] * s_ref[...]


def kernel(x, W1, b1, W2, b2):
    B, C, H, Wd = x.shape
    N = H * Wd
    R = B * C
    k = int(round(N * _PERCENT_T))
    x3 = x.reshape(R, H, Wd)

    rows = _ROWS_PER_BLOCK
    pools = pl.pallas_call(
        functools.partial(_stats_kernel, k=k),
        grid=(R // rows,),
        in_specs=[pl.BlockSpec((rows, H, Wd), lambda i: (i, 0, 0))],
        out_specs=pl.BlockSpec((rows, 2), lambda i: (i, 0)),
        out_shape=jax.ShapeDtypeStruct((R, 2), jnp.float32),
    )(x3)

    avg_pool = pools[:, 0].reshape(B, C)
    max_pool = pools[:, 1].reshape(B, C)

    scale = pl.pallas_call(
        _mlp_kernel,
        out_shape=jax.ShapeDtypeStruct((B, C), jnp.float32),
    )(avg_pool, max_pool, W1, b1.reshape(1, -1), W2, b2.reshape(1, -1))

    scale3 = scale.reshape(R, 1, 1)
    hb = _H_BLK if H % _H_BLK == 0 else H
    y = pl.pallas_call(
        _scale_kernel,
        grid=(R // rows, H // hb),
        in_specs=[
            pl.BlockSpec((rows, hb, Wd), lambda i, j: (i, j, 0)),
            pl.BlockSpec((rows, 1, 1), lambda i, j: (i, 0, 0)),
        ],
        out_specs=pl.BlockSpec((rows, hb, Wd), lambda i, j: (i, j, 0)),
        out_shape=jax.ShapeDtypeStruct((R, H, Wd), jnp.float32),
    )(x3, scale3)

    return y.reshape(B, C, H, Wd)


# speedup vs baseline: 83.6855x; 1.5546x over previous
import functools

import jax
import jax.numpy as jnp
from jax.experimental import pallas as pl

_PERCENT_T = 0.02
_BISECT_ITERS = 12
_ROWS_PER_BLOCK = 8
_H_BLK = 96


def _stats_kernel(x_ref, out_ref, *, k):
    x = x_ref[...]
    rowmax = jnp.max(x, axis=(1, 2), keepdims=True)
    rowmin = jnp.min(x, axis=(1, 2), keepdims=True)
    lo = rowmin - 1.0
    hi = rowmax
    kf = jnp.float32(k)

    def body(_, carry):
        lo, hi = carry
        mid = 0.5 * (lo + hi)
        m = jnp.sum((x > mid).astype(jnp.float32), axis=(1, 2), keepdims=True)
        pred = m >= kf
        lo = jnp.where(pred, mid, lo)
        hi = jnp.where(pred, hi, mid)
        return lo, hi

    lo, hi = jax.lax.fori_loop(0, _BISECT_ITERS, body, (lo, hi))
    mask = x > hi
    m_hi = jnp.sum(mask.astype(jnp.float32), axis=(1, 2), keepdims=True)
    s_hi = jnp.sum(jnp.where(mask, x, 0.0), axis=(1, 2), keepdims=True)
    sum_est = s_hi + (kf - m_hi) * 0.5 * (lo + hi)
    avg = sum_est / kf
    out_ref[...] = jnp.concatenate([avg, rowmax], axis=2)[:, 0, :]


def _mlp_kernel(avg_ref, max_ref, w1_ref, b1_ref, w2_ref, b2_ref, scale_ref):
    w1 = w1_ref[...]
    b1 = b1_ref[...]
    w2 = w2_ref[...]
    b2 = b2_ref[...]

    def mlp(p):
        h = jnp.dot(p, w1.T, preferred_element_type=jnp.float32) + b1
        h = jnp.maximum(h, 0.0)
        return jnp.dot(h, w2.T, preferred_element_type=jnp.float32) + b2

    att = mlp(avg_ref[...]) + mlp(max_ref[...])
    scale_ref[...] = jax.nn.sigmoid(att)


def _scale_kernel(x_ref, s_ref, o_ref):
    o_ref[...] = x_ref[...] * s_ref[...]


def kernel(x, W1, b1, W2, b2):
    B, C, H, Wd = x.shape
    N = H * Wd
    R = B * C
    k = int(round(N * _PERCENT_T))
    x3 = x.reshape(R, H, Wd)

    rows = _ROWS_PER_BLOCK
    pools = pl.pallas_call(
        functools.partial(_stats_kernel, k=k),
        grid=(R // rows,),
        in_specs=[pl.BlockSpec((rows, H, Wd), lambda i: (i, 0, 0))],
        out_specs=pl.BlockSpec((rows, 2), lambda i: (i, 0)),
        out_shape=jax.ShapeDtypeStruct((R, 2), jnp.float32),
    )(x3)

    avg_pool = pools[:, 0].reshape(B, C)
    max_pool = pools[:, 1].reshape(B, C)

    scale = pl.pallas_call(
        _mlp_kernel,
        out_shape=jax.ShapeDtypeStruct((B, C), jnp.float32),
    )(avg_pool, max_pool, W1, b1.reshape(1, -1), W2, b2.reshape(1, -1))

    scale3 = scale.reshape(R, 1, 1)
    hb = _H_BLK if H % _H_BLK == 0 else H
    y = pl.pallas_call(
        _scale_kernel,
        grid=(R // rows, H // hb),
        in_specs=[
            pl.BlockSpec((rows, hb, Wd), lambda i, j: (i, j, 0)),
            pl.BlockSpec((rows, 1, 1), lambda i, j: (i, 0, 0)),
        ],
        out_specs=pl.BlockSpec((rows, hb, Wd), lambda i, j: (i, j, 0)),
        out_shape=jax.ShapeDtypeStruct((R, H, Wd), jnp.float32),
    )(x3, scale3)

    return y.reshape(B, C, H, Wd)
